# baseline (speedup 1.0000x reference)
import functools

import jax
import jax.numpy as jnp
from jax import lax
from jax.experimental import pallas as pl
from jax.experimental.pallas import tpu as pltpu
from jax.experimental.pallas import tpu_sc as plsc

VOCAB = 100000
BATCH = 16384
DIM = 128

_info = plsc.get_sparse_core_info()
NC, NS = _info.num_cores, _info.num_subcores
NW = NC * NS
B_PER_W = BATCH // NW
CHUNK = 128
N_CHUNK = B_PER_W // CHUNK


NBUF = 7


def _body(t0, t1, t2, i0, i1, i2, o0, o1, o2, idx_v, rows_v, gsem, ssem):
    c = lax.axis_index("c")
    s = lax.axis_index("s")
    wid = s * NC + c
    base = wid * B_PER_W
    tbls = (t0, t1, t2)
    idxs = (i0, i1, i2)
    outs = (o0, o1, o2)
    for t in range(3):
        for j in range(N_CHUNK):
            pltpu.sync_copy(
                idxs[t].at[pl.ds(base + j * CHUNK, CHUNK)],
                idx_v.at[t * N_CHUNK + j],
            )
    steps = [
        (tbls[t], t * N_CHUNK + j, outs[t], base + j * CHUNK)
        for t in range(3)
        for j in range(N_CHUNK)
    ]
    n_steps = len(steps)

    def start_gather(k):
        tbl, irow, _, _ = steps[k]
        return pltpu.async_copy(tbl.at[idx_v.at[irow]], rows_v.at[k % NBUF], gsem)

    gat = [None] * n_steps
    sto = [None] * n_steps
    for k in range(min(NBUF, n_steps)):
        gat[k] = start_gather(k)
    for k in range(n_steps):
        gat[k].wait()
        _, _, out, off = steps[k]
        sto[k] = pltpu.async_copy(rows_v.at[k % NBUF], out.at[pl.ds(off, CHUNK)], ssem)
        if k + NBUF < n_steps:
            sto[k].wait()
            gat[k + NBUF] = start_gather(k + NBUF)
    for k in range(max(0, n_steps - NBUF), n_steps):
        sto[k].wait()


@jax.jit
def _run(t0, t1, t2, i0, i1, i2):
    mesh = plsc.VectorSubcoreMesh(core_axis_name="c", subcore_axis_name="s")
    out = jax.ShapeDtypeStruct((BATCH, DIM), jnp.float32)
    k = functools.partial(
        pl.kernel,
        mesh=mesh,
        out_type=(out, out, out),
        scratch_types=[
            pltpu.VMEM((3 * N_CHUNK, CHUNK), jnp.int32),
            pltpu.VMEM((NBUF, CHUNK, DIM), jnp.float32),
            pltpu.SemaphoreType.DMA,
            pltpu.SemaphoreType.DMA,
        ],
    )(_body)
    return k(t0, t1, t2, i0, i1, i2)


def kernel(table_0, table_1, table_2, indices_0, indices_1, indices_2):
    v0, v1, v2 = _run(
        table_0,
        table_1,
        table_2,
        indices_0.astype(jnp.int32),
        indices_1.astype(jnp.int32),
        indices_2.astype(jnp.int32),
    )
    return (v0, v1, v2)


# speedup vs baseline: 1.1244x; 1.1244x over previous
import functools

import jax
import jax.numpy as jnp
from jax import lax
from jax.experimental import pallas as pl
from jax.experimental.pallas import tpu as pltpu
from jax.experimental.pallas import tpu_sc as plsc

VOCAB = 100000
BATCH = 16384
DIM = 128

_info = plsc.get_sparse_core_info()
NC, NS = _info.num_cores, _info.num_subcores
NW = NC * NS
B_PER_W = BATCH // NW
CHUNK = 128
N_CHUNK = B_PER_W // CHUNK


NBUF = 7


def _body(t0, t1, t2, i0, i1, i2, o0, o1, o2, idx_v, rows_v, gsem, ssem, isem):
    c = lax.axis_index("c")
    s = lax.axis_index("s")
    wid = s * NC + c
    base = wid * B_PER_W
    tbls = (t0, t1, t2)
    idxs = (i0, i1, i2)
    outs = (o0, o1, o2)
    icopies = [
        pltpu.async_copy(
            idxs[t].at[wid], idx_v.at[pl.ds(t * N_CHUNK, N_CHUNK)], isem
        )
        for t in range(3)
    ]
    for ic in icopies:
        ic.wait()
    steps = [
        (tbls[t], t * N_CHUNK + j, outs[t], base + j * CHUNK)
        for t in range(3)
        for j in range(N_CHUNK)
    ]
    n_steps = len(steps)

    def start_gather(k):
        tbl, irow, _, _ = steps[k]
        return pltpu.async_copy(tbl.at[idx_v.at[irow]], rows_v.at[k % NBUF], gsem)

    gat = [None] * n_steps
    sto = [None] * n_steps
    for k in range(min(NBUF, n_steps)):
        gat[k] = start_gather(k)
    for k in range(n_steps):
        gat[k].wait()
        _, _, out, off = steps[k]
        sto[k] = pltpu.async_copy(rows_v.at[k % NBUF], out.at[pl.ds(off, CHUNK)], ssem)
        if k + NBUF < n_steps:
            sto[k].wait()
            gat[k + NBUF] = start_gather(k + NBUF)
    for k in range(max(0, n_steps - NBUF), n_steps):
        sto[k].wait()


@jax.jit
def _run(t0, t1, t2, i0, i1, i2):
    mesh = plsc.VectorSubcoreMesh(core_axis_name="c", subcore_axis_name="s")
    out = jax.ShapeDtypeStruct((BATCH, DIM), jnp.float32)
    k = functools.partial(
        pl.kernel,
        mesh=mesh,
        out_type=(out, out, out),
        scratch_types=[
            pltpu.VMEM((3 * N_CHUNK, CHUNK), jnp.int32),
            pltpu.VMEM((NBUF, CHUNK, DIM), jnp.float32),
            pltpu.SemaphoreType.DMA,
            pltpu.SemaphoreType.DMA,
            pltpu.SemaphoreType.DMA,
        ],
    )(_body)
    return k(t0, t1, t2, i0, i1, i2)


def kernel(table_0, table_1, table_2, indices_0, indices_1, indices_2):
    i0 = indices_0.astype(jnp.int32).reshape(NW, N_CHUNK, CHUNK)
    i1 = indices_1.astype(jnp.int32).reshape(NW, N_CHUNK, CHUNK)
    i2 = indices_2.astype(jnp.int32).reshape(NW, N_CHUNK, CHUNK)
    v0, v1, v2 = _run(table_0, table_1, table_2, i0, i1, i2)
    return (v0, v1, v2)


# speedup vs baseline: 1.1259x; 1.0013x over previous
import functools

import jax
import jax.numpy as jnp
from jax import lax
from jax.experimental import pallas as pl
from jax.experimental.pallas import tpu as pltpu
from jax.experimental.pallas import tpu_sc as plsc

VOCAB = 100000
BATCH = 16384
DIM = 128

_info = plsc.get_sparse_core_info()
NC, NS = _info.num_cores, _info.num_subcores
NW = NC * NS
B_PER_W = BATCH // NW
CHUNK = 128
N_CHUNK = B_PER_W // CHUNK


NBUF = 3


def _body(t0, t1, t2, i0, i1, i2, o0, o1, o2, idx_v, rows_v, gsem, ssem, isem):
    c = lax.axis_index("c")
    s = lax.axis_index("s")
    wid = s * NC + c
    base = wid * B_PER_W
    tbls = (t0, t1, t2)
    idxs = (i0, i1, i2)
    outs = (o0, o1, o2)
    icopies = [
        pltpu.async_copy(
            idxs[t].at[wid], idx_v.at[pl.ds(t * N_CHUNK, N_CHUNK)], isem
        )
        for t in range(3)
    ]
    for ic in icopies:
        ic.wait()
    n_super = 3 * N_CHUNK // 2

    def start_gathers(s):
        t, h = s // 2, s % 2
        buf = rows_v.at[s % NBUF]
        return [
            pltpu.async_copy(
                tbls[t].at[idx_v.at[t * N_CHUNK + h * 2 + w]],
                buf.at[pl.ds(w * CHUNK, CHUNK)],
                gsem,
            )
            for w in range(2)
        ]

    gat = [None] * n_super
    sto = [None] * n_super
    for s in range(min(NBUF, n_super)):
        gat[s] = start_gathers(s)
    for s in range(n_super):
        for g in gat[s]:
            g.wait()
        t, h = s // 2, s % 2
        sto[s] = pltpu.async_copy(
            rows_v.at[s % NBUF], outs[t].at[pl.ds(base + h * 2 * CHUNK, 2 * CHUNK)], ssem
        )
        if s + NBUF < n_super:
            sto[s].wait()
            gat[s + NBUF] = start_gathers(s + NBUF)
    for s in range(max(0, n_super - NBUF), n_super):
        sto[s].wait()


@jax.jit
def _run(t0, t1, t2, i0, i1, i2):
    mesh = plsc.VectorSubcoreMesh(core_axis_name="c", subcore_axis_name="s")
    out = jax.ShapeDtypeStruct((BATCH, DIM), jnp.float32)
    k = functools.partial(
        pl.kernel,
        mesh=mesh,
        out_type=(out, out, out),
        scratch_types=[
            pltpu.VMEM((3 * N_CHUNK, CHUNK), jnp.int32),
            pltpu.VMEM((NBUF, 2 * CHUNK, DIM), jnp.float32),
            pltpu.SemaphoreType.DMA,
            pltpu.SemaphoreType.DMA,
            pltpu.SemaphoreType.DMA,
        ],
    )(_body)
    return k(t0, t1, t2, i0, i1, i2)


def kernel(table_0, table_1, table_2, indices_0, indices_1, indices_2):
    i0 = indices_0.astype(jnp.int32).reshape(NW, N_CHUNK, CHUNK)
    i1 = indices_1.astype(jnp.int32).reshape(NW, N_CHUNK, CHUNK)
    i2 = indices_2.astype(jnp.int32).reshape(NW, N_CHUNK, CHUNK)
    v0, v1, v2 = _run(table_0, table_1, table_2, i0, i1, i2)
    return (v0, v1, v2)
